# baseline (speedup 1.0000x reference)
import functools

import jax
import jax.numpy as jnp
from jax import lax
from jax.experimental import pallas as pl
from jax.experimental.pallas import tpu as pltpu
from jax.experimental.pallas import tpu_sc as plsc

N = 1024
B = 32
E = N * (N - 1)
NC = 2
NS = 16
L = 16
NW = NC * NS
ROWS_W = N // NW
IN_CHUNK = ROWS_W * (N - 1)
OUT_CHUNK = ROWS_W * N

_mesh = plsc.VectorSubcoreMesh(core_axis_name="c", subcore_axis_name="s")


@functools.partial(
    pl.kernel,
    out_type=jax.ShapeDtypeStruct((B * N * N,), jnp.float32),
    mesh=_mesh,
    compiler_params=pltpu.CompilerParams(needs_layout_passes=False),
    scratch_types=[
        pltpu.VMEM((IN_CHUNK,), jnp.float32),
        pltpu.VMEM((OUT_CHUNK,), jnp.float32),
        pltpu.SemaphoreType.DMA,
        pltpu.SemaphoreType.DMA,
    ],
)
def _unflatten_sc(g_hbm, out_hbm, in_v, out_v, isem, osem):
    wid = lax.axis_index("s") * NC + lax.axis_index("c")
    r0 = wid * ROWS_W
    lane = lax.iota(jnp.int32, 16)

    def body(b, _):
        g_off = b * E + r0 * (N - 1)
        in_cp = pltpu.make_async_copy(
            g_hbm.at[pl.ds(g_off, IN_CHUNK)], in_v, isem
        )
        in_cp.start()
        in_cp.wait()

        def row_body(i, _):
            r = r0 + i
            jr = lax.div(r, L)
            ibase = i * (N - 1)
            obase = i * N

            col = jr * L + lane
            idx = ibase + col - (col > r).astype(jnp.int32)
            vfix = plsc.load_gather(in_v, [idx])
            vfix = jnp.where(col == r, jnp.float32(0.0), vfix)

            for j in range(N // L):
                off = ibase + j * L - (j > jr).astype(jnp.int32)
                v = in_v[pl.ds(off, L)]
                sel = jax.lax.broadcast(j == jr, (L,))
                out_v[pl.ds(obase + j * L, L)] = jnp.where(sel, vfix, v)
            return 0

        lax.fori_loop(0, ROWS_W, row_body, 0)

        o_off = b * (N * N) + r0 * N
        out_cp = pltpu.make_async_copy(
            out_v, out_hbm.at[pl.ds(o_off, OUT_CHUNK)], osem
        )
        out_cp.start()
        out_cp.wait()
        return 0

    lax.fori_loop(0, B, body, 0)


def kernel(graph):
    flat = graph.reshape(B * E)
    out = _unflatten_sc(flat)
    return out.reshape(B, N, N)
